# Initial kernel scaffold; baseline (speedup 1.0000x reference)
import jax
import jax.numpy as jnp
from jax.experimental import pallas as pl


def kernel(predicted_boxes, predicted_scores, ground_truth_boxes, ground_truth_labels, prior_boxes):
    raise NotImplementedError("write your pallas kernel here")


# speedup vs baseline: 5.2666x; 5.2666x over previous
import jax
import jax.numpy as jnp
from jax.experimental import pallas as pl
from jax.experimental.pallas import tpu as pltpu

B = 32
P = 24564
P_PAD = 24576
R = 192
L = 128
C = 81
NOBJ = 24
THRESHOLD = 0.5
CENTER_VAR = 0.1
SIZE_VAR = 0.2
NEG_POS_RATIO = 3
ALPHA = 1.0


def _loss_kernel(gtb_ref, gtl_ref, pri_ref, box_ref, sco_ref, out_ref):
    pcx = pri_ref[0]
    pcy = pri_ref[1]
    pw = pri_ref[2]
    ph = pri_ref[3]
    px1 = pcx - pw * 0.5
    py1 = pcy - ph * 0.5
    px2 = pcx + pw * 0.5
    py2 = pcy + ph * 0.5
    p_area = (px2 - px1) * (py2 - py1)

    lin = jax.lax.broadcasted_iota(jnp.int32, (R, L), 0) * L + \
        jax.lax.broadcasted_iota(jnp.int32, (R, L), 1)

    best_iou = jnp.full((R, L), -1.0, dtype=jnp.float32)
    best_obj = jnp.zeros((R, L), dtype=jnp.int32)
    gx1s, gy1s, gx2s, gy2s, gls, ppos = [], [], [], [], [], []
    for i in range(NOBJ):
        gx1 = gtb_ref[0, i, 0]
        gy1 = gtb_ref[0, i, 1]
        gx2 = gtb_ref[0, i, 2]
        gy2 = gtb_ref[0, i, 3]
        gx1s.append(gx1)
        gy1s.append(gy1)
        gx2s.append(gx2)
        gy2s.append(gy2)
        gls.append(gtl_ref[0, 0, i])
        g_area = (gx2 - gx1) * (gy2 - gy1)
        iw = jnp.maximum(jnp.minimum(gx2, px2) - jnp.maximum(gx1, px1), 0.0)
        ih = jnp.maximum(jnp.minimum(gy2, py2) - jnp.maximum(gy1, py1), 0.0)
        inter = iw * ih
        iou = inter / (g_area + p_area - inter)
        upd = iou > best_iou
        best_iou = jnp.where(upd, iou, best_iou)
        best_obj = jnp.where(upd, i, best_obj)
        m = jnp.max(iou)
        ppos.append(jnp.min(jnp.where(iou == m, lin, P_PAD)))

    for i in range(NOBJ):
        sel = lin == ppos[i]
        best_obj = jnp.where(sel, i, best_obj)
        best_iou = jnp.where(sel, 1.0, best_iou)

    lab = jnp.zeros((R, L), dtype=jnp.int32)
    mx1 = jnp.zeros((R, L), dtype=jnp.float32)
    my1 = jnp.zeros((R, L), dtype=jnp.float32)
    mx2 = jnp.zeros((R, L), dtype=jnp.float32)
    my2 = jnp.zeros((R, L), dtype=jnp.float32)
    for i in range(NOBJ):
        sel = best_obj == i
        lab = jnp.where(sel, gls[i], lab)
        mx1 = jnp.where(sel, gx1s[i], mx1)
        my1 = jnp.where(sel, gy1s[i], my1)
        mx2 = jnp.where(sel, gx2s[i], mx2)
        my2 = jnp.where(sel, gy2s[i], my2)

    lab = jnp.where(best_iou < THRESHOLD, 0, lab)
    pos = lab != 0
    posf = pos.astype(jnp.float32)
    n_pos = jnp.sum(pos.astype(jnp.int32))

    mcx = (mx1 + mx2) * 0.5
    mcy = (my1 + my2) * 0.5
    mw = mx2 - mx1
    mh = my2 - my1
    gcx = (mcx - pcx) / (pw * CENTER_VAR)
    gcy = (mcy - pcy) / (ph * CENTER_VAR)
    gw = jnp.log(mw / pw) / SIZE_VAR
    gh = jnp.log(mh / ph) / SIZE_VAR
    loc_sum = (
        jnp.sum(jnp.abs(box_ref[0, 0] - gcx) * posf)
        + jnp.sum(jnp.abs(box_ref[0, 1] - gcy) * posf)
        + jnp.sum(jnp.abs(box_ref[0, 2] - gw) * posf)
        + jnp.sum(jnp.abs(box_ref[0, 3] - gh) * posf)
    )

    s = sco_ref[0]
    smax = jnp.max(s, axis=-1)
    sexp = jnp.sum(jnp.exp(s - smax[:, :, None]), axis=-1)
    lse = smax + jnp.log(sexp)
    cidx = jax.lax.broadcasted_iota(jnp.int32, (R, L, C), 2)
    s_at = jnp.sum(jnp.where(cidx == lab[:, :, None], s, 0.0), axis=-1)
    ce = lse - s_at

    conf_pos = jnp.sum(ce * posf)

    valid = lin < P
    ce_neg = jnp.maximum(jnp.where(pos | (~valid), 0.0, ce), 0.0)

    k = jnp.minimum(NEG_POS_RATIO * n_pos, P)
    bits = jax.lax.bitcast_convert_type(ce_neg, jnp.int32)

    def bs_body(i, ans):
        cand = ans | jax.lax.shift_left(jnp.int32(1), 30 - i)
        cnt = jnp.sum((bits >= cand).astype(jnp.int32))
        return jax.lax.select(cnt >= k, cand, ans)

    ans = jax.lax.fori_loop(0, 31, bs_body, jnp.int32(0))
    thr = jax.lax.bitcast_convert_type(ans, jnp.float32)
    gt_mask = bits > ans
    cnt_gt = jnp.sum(gt_mask.astype(jnp.int32))
    sum_gt = jnp.sum(jnp.where(gt_mask, ce_neg, 0.0))
    conf_hard = jnp.where(
        k > 0, sum_gt + (k - cnt_gt).astype(jnp.float32) * thr, 0.0
    )

    out_ref[0, 0, 0] = loc_sum
    out_ref[0, 0, 1] = conf_pos
    out_ref[0, 0, 2] = conf_hard
    out_ref[0, 0, 3] = n_pos.astype(jnp.float32)


def kernel(predicted_boxes, predicted_scores, ground_truth_boxes,
           ground_truth_labels, prior_boxes):
    pad = P_PAD - P
    pb = jnp.pad(predicted_boxes, ((0, 0), (0, pad), (0, 0)))
    pb = jnp.transpose(pb, (0, 2, 1)).reshape(B, 4, R, L)
    ps = jnp.pad(predicted_scores, ((0, 0), (0, pad), (0, 0)))
    ps = ps.reshape(B, R, L, C)
    pad_rows = jnp.broadcast_to(
        jnp.array([-10.0, -10.0, 1.0, 1.0], dtype=jnp.float32), (pad, 4))
    pr = jnp.concatenate([prior_boxes, pad_rows], axis=0)
    pr = jnp.transpose(pr).reshape(4, R, L)
    gtl = ground_truth_labels.astype(jnp.int32).reshape(B, 1, NOBJ)

    out = pl.pallas_call(
        _loss_kernel,
        grid=(B,),
        in_specs=[
            pl.BlockSpec((1, NOBJ, 4), lambda b: (b, 0, 0),
                         memory_space=pltpu.SMEM),
            pl.BlockSpec((1, 1, NOBJ), lambda b: (b, 0, 0),
                         memory_space=pltpu.SMEM),
            pl.BlockSpec((4, R, L), lambda b: (0, 0, 0)),
            pl.BlockSpec((1, 4, R, L), lambda b: (b, 0, 0, 0)),
            pl.BlockSpec((1, R, L, C), lambda b: (b, 0, 0, 0)),
        ],
        out_specs=pl.BlockSpec((1, 1, 4), lambda b: (b, 0, 0),
                               memory_space=pltpu.SMEM),
        out_shape=jax.ShapeDtypeStruct((B, 1, 4), jnp.float32),
        compiler_params=pltpu.CompilerParams(
            vmem_limit_bytes=110 * 1024 * 1024),
    )(ground_truth_boxes, gtl, pr, pb, ps)

    loc_sum = jnp.sum(out[:, 0, 0])
    conf_pos = jnp.sum(out[:, 0, 1])
    conf_hard = jnp.sum(out[:, 0, 2])
    n_pos = jnp.sum(out[:, 0, 3])
    location_loss = loc_sum / (n_pos * 4.0)
    confidence_loss = (conf_hard + conf_pos) / n_pos
    return (confidence_loss, ALPHA * location_loss)


# speedup vs baseline: 6.9763x; 1.3246x over previous
import jax
import jax.numpy as jnp
from jax.experimental import pallas as pl
from jax.experimental.pallas import tpu as pltpu

B = 32
P = 24564
P_PAD = 24576
R = 192
L = 128
C = 81
NOBJ = 24
THRESHOLD = 0.5
CENTER_VAR = 0.1
SIZE_VAR = 0.2
NEG_POS_RATIO = 3
ALPHA = 1.0


def _match_kernel(gtb_ref, gtl_ref, pri_ref, box_ref, lab_ref, out_ref):
    pcx = pri_ref[0]
    pcy = pri_ref[1]
    pw = pri_ref[2]
    ph = pri_ref[3]
    px1 = pcx - pw * 0.5
    py1 = pcy - ph * 0.5
    px2 = pcx + pw * 0.5
    py2 = pcy + ph * 0.5
    p_area = (px2 - px1) * (py2 - py1)

    lin = jax.lax.broadcasted_iota(jnp.int32, (R, L), 0) * L + \
        jax.lax.broadcasted_iota(jnp.int32, (R, L), 1)

    best_iou = jnp.full((R, L), -1.0, dtype=jnp.float32)
    best_obj = jnp.zeros((R, L), dtype=jnp.int32)
    gx1s, gy1s, gx2s, gy2s, gls, ppos = [], [], [], [], [], []
    for i in range(NOBJ):
        gx1 = gtb_ref[0, i, 0]
        gy1 = gtb_ref[0, i, 1]
        gx2 = gtb_ref[0, i, 2]
        gy2 = gtb_ref[0, i, 3]
        gx1s.append(gx1)
        gy1s.append(gy1)
        gx2s.append(gx2)
        gy2s.append(gy2)
        gls.append(gtl_ref[0, 0, i])
        g_area = (gx2 - gx1) * (gy2 - gy1)
        iw = jnp.maximum(jnp.minimum(gx2, px2) - jnp.maximum(gx1, px1), 0.0)
        ih = jnp.maximum(jnp.minimum(gy2, py2) - jnp.maximum(gy1, py1), 0.0)
        inter = iw * ih
        iou = inter / (g_area + p_area - inter)
        upd = iou > best_iou
        best_iou = jnp.where(upd, iou, best_iou)
        best_obj = jnp.where(upd, i, best_obj)
        m = jnp.max(iou)
        ppos.append(jnp.min(jnp.where(iou == m, lin, P_PAD)))

    for i in range(NOBJ):
        sel = lin == ppos[i]
        best_obj = jnp.where(sel, i, best_obj)
        best_iou = jnp.where(sel, 1.0, best_iou)

    lab = jnp.zeros((R, L), dtype=jnp.int32)
    mx1 = jnp.zeros((R, L), dtype=jnp.float32)
    my1 = jnp.zeros((R, L), dtype=jnp.float32)
    mx2 = jnp.zeros((R, L), dtype=jnp.float32)
    my2 = jnp.zeros((R, L), dtype=jnp.float32)
    for i in range(NOBJ):
        sel = best_obj == i
        lab = jnp.where(sel, gls[i], lab)
        mx1 = jnp.where(sel, gx1s[i], mx1)
        my1 = jnp.where(sel, gy1s[i], my1)
        mx2 = jnp.where(sel, gx2s[i], mx2)
        my2 = jnp.where(sel, gy2s[i], my2)

    lab = jnp.where(best_iou < THRESHOLD, 0, lab)
    lab_ref[0] = lab
    pos = lab != 0
    posf = pos.astype(jnp.float32)
    n_pos = jnp.sum(pos.astype(jnp.int32))

    mcx = (mx1 + mx2) * 0.5
    mcy = (my1 + my2) * 0.5
    mw = mx2 - mx1
    mh = my2 - my1
    gcx = (mcx - pcx) / (pw * CENTER_VAR)
    gcy = (mcy - pcy) / (ph * CENTER_VAR)
    gw = jnp.log(mw / pw) / SIZE_VAR
    gh = jnp.log(mh / ph) / SIZE_VAR
    loc_sum = (
        jnp.sum(jnp.abs(box_ref[0, 0] - gcx) * posf)
        + jnp.sum(jnp.abs(box_ref[0, 1] - gcy) * posf)
        + jnp.sum(jnp.abs(box_ref[0, 2] - gw) * posf)
        + jnp.sum(jnp.abs(box_ref[0, 3] - gh) * posf)
    )
    out_ref[0, 0, 0] = loc_sum
    out_ref[0, 0, 1] = n_pos.astype(jnp.float32)


CH = 2048
NCH = 12


def _ce_kernel(lab_ref, sco_ref, ceneg_ref, out_ref):
    j = pl.program_id(1)
    lab = lab_ref[0]
    s = sco_ref[0]
    smax = jnp.max(s, axis=-1, keepdims=True)
    sexp = jnp.sum(jnp.exp(s - smax), axis=-1, keepdims=True)
    lse = smax + jnp.log(sexp)
    cidx = jax.lax.broadcasted_iota(jnp.int32, (CH, C), 1)
    s_at = jnp.sum(jnp.where(cidx == lab, s, 0.0), axis=-1, keepdims=True)
    ce = lse - s_at
    pos = lab != 0
    rows = jax.lax.broadcasted_iota(jnp.int32, (CH, 1), 0) + j * CH
    valid = rows < P
    conf_pos = jnp.sum(jnp.where(pos & valid, ce, 0.0))
    ceneg_ref[0] = jnp.maximum(jnp.where(pos, 0.0, ce), 0.0)

    @pl.when(j == 0)
    def _():
        out_ref[0, 0, 0] = 0.0

    out_ref[0, 0, 0] += conf_pos


def _sel_kernel(k_ref, ceneg_ref, out_ref):
    ce_neg = ceneg_ref[0]
    k = k_ref[0, 0, 0]

    bits = jax.lax.bitcast_convert_type(ce_neg, jnp.int32)

    def bs_body(i, ans):
        cand = ans | jax.lax.shift_left(jnp.int32(1), 30 - i)
        cnt = jnp.sum((bits >= cand).astype(jnp.int32))
        return jax.lax.select(cnt >= k, cand, ans)

    ans = jax.lax.fori_loop(0, 31, bs_body, jnp.int32(0))
    thr = jax.lax.bitcast_convert_type(ans, jnp.float32)
    gt_mask = bits > ans
    cnt_gt = jnp.sum(gt_mask.astype(jnp.int32))
    sum_gt = jnp.sum(jnp.where(gt_mask, ce_neg, 0.0))
    out_ref[0, 0, 0] = jnp.where(
        k > 0, sum_gt + (k - cnt_gt).astype(jnp.float32) * thr, 0.0)


def kernel(predicted_boxes, predicted_scores, ground_truth_boxes,
           ground_truth_labels, prior_boxes):
    pad = P_PAD - P
    pb = jnp.pad(predicted_boxes, ((0, 0), (0, pad), (0, 0)))
    pb = jnp.transpose(pb, (0, 2, 1)).reshape(B, 4, R, L)
    pad_rows = jnp.broadcast_to(
        jnp.array([-10.0, -10.0, 1.0, 1.0], dtype=jnp.float32), (pad, 4))
    pr = jnp.concatenate([prior_boxes, pad_rows], axis=0)
    pr = jnp.transpose(pr).reshape(4, R, L)
    gtl = ground_truth_labels.astype(jnp.int32).reshape(B, 1, NOBJ)

    labels_rl, m_out = pl.pallas_call(
        _match_kernel,
        grid=(B,),
        in_specs=[
            pl.BlockSpec((1, NOBJ, 4), lambda b: (b, 0, 0),
                         memory_space=pltpu.SMEM),
            pl.BlockSpec((1, 1, NOBJ), lambda b: (b, 0, 0),
                         memory_space=pltpu.SMEM),
            pl.BlockSpec((4, R, L), lambda b: (0, 0, 0)),
            pl.BlockSpec((1, 4, R, L), lambda b: (b, 0, 0, 0)),
        ],
        out_specs=[
            pl.BlockSpec((1, R, L), lambda b: (b, 0, 0)),
            pl.BlockSpec((1, 1, 2), lambda b: (b, 0, 0),
                         memory_space=pltpu.SMEM),
        ],
        out_shape=[
            jax.ShapeDtypeStruct((B, R, L), jnp.int32),
            jax.ShapeDtypeStruct((B, 1, 2), jnp.float32),
        ],
    )(ground_truth_boxes, gtl, pr, pb)

    lab_flat = labels_rl.reshape(B, P_PAD)[:, :P].reshape(B, P, 1)

    ce_neg_flat, ce_out = pl.pallas_call(
        _ce_kernel,
        grid=(B, NCH),
        in_specs=[
            pl.BlockSpec((1, CH, 1), lambda b, j: (b, j, 0)),
            pl.BlockSpec((1, CH, C), lambda b, j: (b, j, 0)),
        ],
        out_specs=[
            pl.BlockSpec((1, CH, 1), lambda b, j: (b, j, 0)),
            pl.BlockSpec((1, 1, 1), lambda b, j: (b, 0, 0),
                         memory_space=pltpu.SMEM),
        ],
        out_shape=[
            jax.ShapeDtypeStruct((B, P, 1), jnp.float32),
            jax.ShapeDtypeStruct((B, 1, 1), jnp.float32),
        ],
        compiler_params=pltpu.CompilerParams(
            vmem_limit_bytes=110 * 1024 * 1024),
    )(lab_flat, predicted_scores)

    n_pos_img = m_out[:, 0, 1]
    k_img = jnp.minimum(
        NEG_POS_RATIO * n_pos_img.astype(jnp.int32), P).reshape(B, 1, 1)
    ce_neg_rl = jnp.pad(
        ce_neg_flat.reshape(B, P), ((0, 0), (0, pad))).reshape(B, R, L)

    conf_hard_img = pl.pallas_call(
        _sel_kernel,
        grid=(B,),
        in_specs=[
            pl.BlockSpec((1, 1, 1), lambda b: (b, 0, 0),
                         memory_space=pltpu.SMEM),
            pl.BlockSpec((1, R, L), lambda b: (b, 0, 0)),
        ],
        out_specs=pl.BlockSpec((1, 1, 1), lambda b: (b, 0, 0),
                               memory_space=pltpu.SMEM),
        out_shape=jax.ShapeDtypeStruct((B, 1, 1), jnp.float32),
    )(k_img, ce_neg_rl)

    loc_sum = jnp.sum(m_out[:, 0, 0])
    n_pos = jnp.sum(n_pos_img)
    conf_pos = jnp.sum(ce_out[:, 0, 0])
    conf_hard = jnp.sum(conf_hard_img[:, 0, 0])
    location_loss = loc_sum / (n_pos * 4.0)
    confidence_loss = (conf_hard + conf_pos) / n_pos
    return (confidence_loss, ALPHA * location_loss)
